# baseline (speedup 1.0000x reference)
import functools

import jax
import jax.numpy as jnp
from jax import lax
from jax.experimental import pallas as pl
from jax.experimental.pallas import tpu as pltpu
from jax.experimental.pallas import tpu_sc as plsc

NC = 2
NS = 16
NW = NC * NS
CAP = 14336
STRIP = 4096
COLW = 128
PDIM = 128
RB = 8


def _rank_kernel(n_rows: int):
    n_chunks = n_rows // 128
    grid = n_chunks // RB

    def body(x_ref, rank_ref, hist_ref, acc):
        @pl.when(pl.program_id(0) == 0)
        def _():
            acc[...] = jnp.zeros((32, 128), jnp.int32)

        r8 = x_ref[...]
        b8 = (r8 >> 7) & 31
        iota32 = lax.broadcasted_iota(jnp.int32, (32, 128), 0)
        for i in range(RB):
            brow = b8[i:i + 1, :]
            oh = (iota32 == brow).astype(jnp.int32)
            pref = oh
            for d in (1, 2, 4, 8, 16, 32, 64):
                shifted = jnp.concatenate(
                    [jnp.zeros((32, d), jnp.int32), pref[:, :-d]], axis=1)
                pref = pref + shifted
            exc = pref - oh
            intra = jnp.sum(oh * exc, axis=0, keepdims=True)
            base = jnp.sum(oh * acc[:, 0:1], axis=0, keepdims=True)
            rank_ref[i:i + 1, :] = base + intra
            acc[...] = acc[...] + jnp.sum(oh, axis=1, keepdims=True)

        @pl.when(pl.program_id(0) == grid - 1)
        def _():
            hist_ref[...] = acc[...]

    return pl.pallas_call(
        body,
        grid=(grid,),
        in_specs=[pl.BlockSpec((RB, 128), lambda i: (i, 0))],
        out_specs=[
            pl.BlockSpec((RB, 128), lambda i: (i, 0)),
            pl.BlockSpec((32, 128), lambda i: (0, 0)),
        ],
        out_shape=[
            jax.ShapeDtypeStruct((n_chunks, 128), jnp.int32),
            jax.ShapeDtypeStruct((32, 128), jnp.int32),
        ],
        scratch_shapes=[pltpu.VMEM((32, 128), jnp.int32)],
    )


@functools.lru_cache(maxsize=None)
def _build(n_rows: int, vocab: int, dim: int):
    n_cols = (vocab + COLW - 1) // COLW
    last_col = n_cols - 1
    n_strips = n_rows // STRIP
    n_blk = (n_cols + NW - 1) // NW
    dump = n_rows
    out_rows = n_rows + 64
    trash = CAP + 8

    mesh = plsc.VectorSubcoreMesh(core_axis_name="c", subcore_axis_name="s")

    scratch = [
        pltpu.VMEM((STRIP,), jnp.int32),
        pltpu.VMEM((STRIP,), jnp.int32),
        pltpu.VMEM((STRIP,), jnp.int32),
        pltpu.VMEM((STRIP,), jnp.int32),
        pltpu.VMEM((CAP + 32,), jnp.int32),
        pltpu.VMEM((CAP + 32,), jnp.int32),
        pltpu.VMEM((CAP + 32,), jnp.int32),
        pltpu.VMEM((CAP + 32,), jnp.int32),
        pltpu.VMEM((256,), jnp.int32),
        pltpu.VMEM((256,), jnp.int32),
        pltpu.VMEM((256,), jnp.int32),
        pltpu.VMEM((32, 128), jnp.int32),
        pltpu.VMEM((64, PDIM), jnp.float32),
        pltpu.VMEM((64, PDIM), jnp.float32),
        pltpu.VMEM((64, PDIM), jnp.float32),
        pltpu.VMEM((64, PDIM), jnp.float32),
        pltpu.VMEM((1, 64), jnp.int32),
        pltpu.VMEM((1, 64), jnp.int32),
        pltpu.SemaphoreType.DMA,
        pltpu.SemaphoreType.DMA,
        pltpu.SemaphoreType.DMA,
        pltpu.SemaphoreType.DMA,
        pltpu.SemaphoreType.DMA,
        pltpu.SemaphoreType.DMA,
        pltpu.SemaphoreType.DMA,
        pltpu.SemaphoreType.DMA,
        pltpu.SemaphoreType.DMA,
    ]

    @functools.partial(
        pl.kernel,
        out_type=jax.ShapeDtypeStruct((out_rows, PDIM), jnp.float32),
        mesh=mesh,
        scratch_types=scratch,
        compiler_params=pltpu.CompilerParams(
            use_tc_tiling_on_sc=True, needs_layout_passes=False),
    )
    def emb(idx_hbm, rank_hbm, histtc_hbm, wt_hbm, wtail_hbm, out_hbm,
            xs0, xs1, ks0, ks1, sorted_r, sorted_i, grouped_r, grouped_i,
            hist, bstart, cursors, histv,
            stg0, stg1, row0, row1, pos0, pos1,
            xsem0, xsem1, ksem0, ksem1, gsem0, gsem1, osem0, osem1, hsem):
        xs = (xs0, xs1)
        ks = (ks0, ks1)
        xsem = (xsem0, xsem1)
        ksem = (ksem0, ksem1)
        stg = (stg0, stg1)
        gsem = (gsem0, gsem1)
        rowb = (row0, row1)
        posb = (pos0, pos1)
        osem = (osem0, osem1)

        wid = lax.axis_index("s") * NC + lax.axis_index("c")
        iota = lax.iota(jnp.int32, 16)
        iq = [iota + 16 * q for q in range(4)]
        ones = jnp.full((16,), 1, jnp.int32)
        zeros = jnp.full((16,), 0, jnp.int32)
        dump16 = jnp.full((16,), dump, jnp.int32)

        cp = pltpu.async_copy(histtc_hbm, histv, hsem)

        def x_dma(s, b):
            pltpu.async_copy(
                idx_hbm.at[pl.ds(s * STRIP, STRIP)], xs[b], xsem[b])
            pltpu.async_copy(
                rank_hbm.at[pl.ds(s * STRIP, STRIP)], ks[b], ksem[b])

        def x_wait(b):
            pltpu.make_async_copy(
                idx_hbm.at[pl.ds(0, STRIP)], xs[b], xsem[b]).wait()
            pltpu.make_async_copy(
                rank_hbm.at[pl.ds(0, STRIP)], ks[b], ksem[b]).wait()

        x_dma(0, 0)
        x_dma(1, 1)
        cp.wait()
        hv = histv[wid, pl.ds(0, 16)]
        cnt = jnp.minimum(hv[0], CAP)

        def pha_strip(s, b, carry):
            x_wait(b)

            def rowk(j4, carry):
                for jj in range(4):
                    j = j4 * 4 + jj
                    r16 = xs[b][pl.ds(j * 16, 16)]
                    k16 = ks[b][pl.ds(j * 16, 16)]
                    m = ((r16 >> 7) & 31) == wid
                    pos16 = jnp.where(m, jnp.minimum(k16, trash), trash)
                    plsc.store_scatter(sorted_r, [pos16], r16)
                    i16 = iota + (s * STRIP + j * 16)
                    plsc.store_scatter(sorted_i, [pos16], i16)
                return carry

            return lax.fori_loop(0, STRIP // 64, rowk, carry)

        def pha2(s2, carry):
            s = s2 * 2
            carry = pha_strip(s, 0, carry)

            @pl.when(s + 2 < n_strips)
            def _():
                x_dma(s + 2, 0)

            carry = pha_strip(s + 1, 1, carry)

            @pl.when(s + 3 < n_strips)
            def _():
                x_dma(s + 3, 1)

            return carry

        lax.fori_loop(0, n_strips // 2, pha2, 0)

        for v in range(16):
            hist[pl.ds(v * 16, 16)] = zeros

        nv = (cnt + 15) >> 4

        def hpass(v, carry):
            r16 = sorted_r[pl.ds(v * 16, 16)]
            valid = (iota + v * 16) < cnt
            b16 = jnp.where(valid, (r16 >> 12) & 255, 255)
            plsc.addupdate_scatter(hist, [b16], ones)
            return carry

        lax.fori_loop(0, nv, hpass, 0)

        carry = 0
        for v in range(16):
            h16 = hist[pl.ds(v * 16, 16)]
            c16 = plsc.cumsum(h16)
            e16 = (c16 - h16) + carry
            bstart[pl.ds(v * 16, 16)] = e16
            cursors[pl.ds(v * 16, 16)] = e16
            carry = carry + c16[15]

        def regroup(v, carry):
            r16 = sorted_r[pl.ds(v * 16, 16)]
            i16 = sorted_i[pl.ds(v * 16, 16)]
            valid = (iota + v * 16) < cnt
            b16 = jnp.where(valid, (r16 >> 12) & 255, 255)
            pos16 = zeros
            for u in range(16):
                bu = lax.full((16,), b16[u], jnp.int32)
                cv = plsc.load_gather(cursors, [bu])
                plsc.store_scatter(cursors, [bu], cv + 1)
                pos16 = jnp.where(iota == u, cv[0], pos16)
            pos16 = jnp.minimum(pos16, trash)
            pos16 = jnp.where(valid, pos16, trash)
            plsc.store_scatter(grouped_r, [pos16], r16)
            plsc.store_scatter(grouped_i, [pos16], i16)
            return carry

        lax.fori_loop(0, nv, regroup, 0)

        def stage_dma(k, b):
            col = k * NW + wid

            @pl.when(col < last_col)
            def _():
                for cblk in range(8):
                    pltpu.async_copy(
                        wt_hbm.at[pl.ds(cblk * 8, 8), pl.ds(col * COLW, COLW)],
                        stg[b].at[pl.ds(cblk * 8, 8)], gsem[b])

            @pl.when(col == last_col)
            def _():
                pltpu.async_copy(wtail_hbm, stg[b], gsem[b])

        def stage_wait(b):
            pltpu.make_async_copy(
                wt_hbm.at[:, pl.ds(0, COLW)], stg[b], gsem[b]).wait()

        def scatter(p):
            return pltpu.async_copy(
                rowb[p], out_hbm.at[posb[p].at[0]], osem[p])

        def scatter_wait(p):
            pltpu.make_async_copy(
                rowb[p], out_hbm.at[posb[p].at[0]], osem[p]).wait()

        for p in range(2):
            for v in range(4):
                posb[p][0, pl.ds(v * 16, 16)] = dump16 + iota + 16 * v
            scatter(p)

        def extract_vec(e, hi, start, b, p, v):
            r16 = grouped_r[pl.ds(e, 16)]
            i16 = grouped_i[pl.ds(e, 16)]
            m = (iota + e) < hi
            posb[p][0, pl.ds(v * 16, 16)] = jnp.where(
                m, i16, dump16 + iota + 16 * v)
            l16 = (r16 - start) & (COLW - 1)
            for u in range(16):
                lu = lax.full((16,), l16[u], jnp.int32)
                for q in range(4):
                    g = plsc.load_gather(stg[b], [iq[q], lu])
                    rowb[p][v * 16 + u, pl.ds(q * 16, 16)] = g

        def process_block(k, b):
            col = k * NW + wid
            start = col * COLW
            bidx = jnp.where(iota == 0, k, k + 1)
            bg = plsc.load_gather(bstart, [bidx])
            lo = bg[0]
            hi = bg[1]
            nq = (hi - lo + 63) >> 6

            def q2(t, carry):
                for p in range(2):
                    qi = t * 2 + p

                    @pl.when(qi < nq)
                    def _():
                        scatter_wait(p)
                        for v in range(4):
                            e = lo + 16 * (qi * 4 + v)

                            @pl.when(e < hi)
                            def _():
                                extract_vec(e, hi, start, b, p, v)

                        scatter(p)

                return carry

            lax.fori_loop(0, (nq + 1) >> 1, q2, 0)

        stage_dma(0, 0)

        def blk2(j, carry):
            k = j * 2
            stage_wait(0)
            stage_dma(k + 1, 1)
            process_block(k, 0)
            stage_wait(1)

            @pl.when((k + 2 < n_blk - 1) | ((k + 2) * NW + wid < n_cols))
            def _():
                stage_dma(k + 2, 0)

            process_block(k + 1, 1)
            return carry

        lax.fori_loop(0, (n_blk - 1) // 2, blk2, 0)

        k_last = n_blk - 1

        @pl.when(k_last * NW + wid < n_cols)
        def _():
            stage_wait(0)
            process_block(k_last, 0)

        scatter_wait(0)
        scatter_wait(1)

    rank_fn = _rank_kernel(n_rows)

    def run(x, weight):
        idx2 = x.reshape(n_rows // 128, 128).astype(jnp.int32)
        rank2, hist_tc = rank_fn(idx2)
        wt = weight.T
        tail0 = last_col * COLW
        wtail = jnp.pad(
            weight[tail0:].T, ((0, 0), (0, COLW - (vocab - tail0))))
        out3 = emb(idx2.reshape(n_rows), rank2.reshape(n_rows),
                   hist_tc, wt, wtail)
        return out3[:n_rows, :dim]

    return run


def kernel(x, weight):
    b, h = x.shape
    v, d = weight.shape
    run = _build(b * h, v, d)
    return run(x, weight).reshape(b, h, d)


# speedup vs baseline: 1.8801x; 1.8801x over previous
import functools

import jax
import jax.numpy as jnp
from jax import lax
from jax.experimental import pallas as pl
from jax.experimental.pallas import tpu as pltpu
from jax.experimental.pallas import tpu_sc as plsc

NC = 2
NS = 16
NW = NC * NS
CHUNK = 128
NBUF = 4


@functools.lru_cache(maxsize=None)
def _build(n_rows: int, dim: int):
    rows_per_w = n_rows // NW
    n_chunks = rows_per_w // CHUNK
    n_rounds = n_chunks // NBUF
    assert n_rows % (NW * CHUNK * NBUF * 2) == 0
    total_chunk_rows = n_rows // CHUNK

    mesh = plsc.VectorSubcoreMesh(core_axis_name="c", subcore_axis_name="s")

    scratch = [pltpu.VMEM((n_chunks, CHUNK), jnp.int32)]
    scratch += [pltpu.VMEM((CHUNK, dim), jnp.float32) for _ in range(2 * NBUF)]
    scratch += [pltpu.SemaphoreType.DMA for _ in range(2 * NBUF)]
    scratch += [pltpu.SemaphoreType.DMA for _ in range(2 * NBUF)]

    @functools.partial(
        pl.kernel,
        out_type=jax.ShapeDtypeStruct((n_rows, dim), jnp.float32),
        mesh=mesh,
        scratch_types=scratch,
        compiler_params=pltpu.CompilerParams(use_tc_tiling_on_sc=False),
    )
    def emb(idx_hbm, table_hbm, out_hbm, idx_v, *rest):
        bufs = [[rest[2 * b + p] for p in range(2)] for b in range(NBUF)]
        o = 2 * NBUF
        gsem = [[rest[o + 2 * b + p] for p in range(2)] for b in range(NBUF)]
        o = 4 * NBUF
        psem = [[rest[o + 2 * b + p] for p in range(2)] for b in range(NBUF)]

        wid = lax.axis_index("s") * NC + lax.axis_index("c")
        base = wid * rows_per_w
        chunk0 = wid * n_chunks

        pltpu.sync_copy(idx_hbm.at[pl.ds(chunk0, n_chunks)], idx_v)

        def gather(g, b, p):
            return pltpu.async_copy(
                table_hbm.at[idx_v.at[g]], bufs[b][p], gsem[b][p])

        def gather_wait(g, b, p):
            pltpu.make_async_copy(
                table_hbm.at[idx_v.at[g]], bufs[b][p], gsem[b][p]).wait()

        def put(g, b, p):
            return pltpu.async_copy(
                bufs[b][p], out_hbm.at[pl.ds(base + g * CHUNK, CHUNK)],
                psem[b][p])

        def put_wait(g, b, p):
            pltpu.make_async_copy(
                bufs[b][p], out_hbm.at[pl.ds(base + g * CHUNK, CHUNK)],
                psem[b][p]).wait()

        for b in range(NBUF):
            gather(b, b, 0)
        for b in range(NBUF):
            gather_wait(b, b, 0)
            put(b, b, 0)
            gather(NBUF + b, b, 1)

        def round_(r, p):
            for b in range(NBUF):
                g = r * NBUF + b
                gather_wait(g, b, p)
                put(g, b, p)
                put_wait(g - NBUF, b, 1 - p)
                gather(g + NBUF, b, 1 - p)

        def body(k, carry):
            round_(2 * k + 1, 1)
            round_(2 * k + 2, 0)
            return carry

        lax.fori_loop(0, (n_rounds - 2) // 2, body, 0)

        r = n_rounds - 1
        for b in range(NBUF):
            g = r * NBUF + b
            gather_wait(g, b, 1)
            put(g, b, 1)
            put_wait(g - NBUF, b, 0)
        for b in range(NBUF):
            put_wait(r * NBUF + b, b, 1)

    def run(x, weight):
        idx = x.reshape(total_chunk_rows, CHUNK).astype(jnp.int32)
        out = emb(idx, weight)
        return out

    return run


def kernel(x, weight):
    b, h = x.shape
    v, d = weight.shape
    run = _build(b * h, d)
    return run(x, weight).reshape(b, h, d)
